# baseline (speedup 1.0000x reference)
import jax
import jax.numpy as jnp
from jax import lax
from jax.experimental import pallas as pl
from jax.experimental.pallas import tpu as pltpu

TOPK = 512
PROMPT_WIN = 256
NHI = 16
TS = 512
TO = 512


def _compute_sel(pb, xs):
    D = xs.shape[1]
    rows = lax.broadcasted_iota(jnp.int32, (PROMPT_WIN, 1), 0)
    rmask = (rows <= pb).astype(jnp.float32)
    s = jnp.sum(xs * rmask, axis=0, keepdims=True)
    a = jnp.abs(s) / (pb + 1).astype(jnp.float32)
    ab = lax.bitcast_convert_type(a, jnp.int32)

    def bs_body(_, carry):
        lo, hi = carry
        mid = lo + (hi - lo) // 2
        ge = jnp.sum((ab >= mid).astype(jnp.int32))
        take = ge >= TOPK
        return jnp.where(take, mid, lo), jnp.where(take, hi, mid)

    lo, _ = lax.fori_loop(
        0, 31, bs_body, (jnp.int32(0), jnp.int32(0x7F800001)))
    v = lo
    gt = ab > v
    c_gt = jnp.sum(gt.astype(jnp.int32))
    r = TOPK - c_gt
    eq = ab == v
    idx = lax.broadcasted_iota(jnp.int32, (1, D), 1)

    def bs2_body(_, carry):
        lo2, hi2 = carry
        mid = lo2 + (hi2 - lo2) // 2
        g = jnp.sum((eq & (idx <= mid)).astype(jnp.int32))
        ok = g >= r
        lo2n = jnp.where(ok, lo2, mid)
        hi2n = jnp.where(ok, mid, hi2)
        valid = (hi2 - lo2) > 1
        return (jnp.where(valid, lo2n, lo2), jnp.where(valid, hi2n, hi2))

    _, I = lax.fori_loop(0, 11, bs2_body, (jnp.int32(-1), jnp.int32(D - 1)))
    return (gt | (eq & (idx <= I))).astype(jnp.float32)


def _scatter_mask(sel, perm_row, m_ref):
    D = perm_row.shape[1]
    nlo = D // NHI
    permc = jnp.transpose(perm_row)
    selc = jnp.transpose(sel)
    hids = lax.broadcasted_iota(jnp.int32, (1, NHI), 1)
    lids = lax.broadcasted_iota(jnp.int32, (1, nlo), 1)
    asel = jnp.where((permc // nlo) == hids, selc, 0.0).astype(jnp.bfloat16)
    b = ((permc % nlo) == lids).astype(jnp.bfloat16)
    mm = lax.dot_general(
        asel, b, (((0,), (0,)), ((), ())),
        preferred_element_type=jnp.float32)
    for h in range(NHI):
        m_ref[0:1, h * nlo:(h + 1) * nlo] = mm[h:h + 1, :]


def _fused_kernel(pb_ref, perm_ref, xp_ref, x_any, w_any, out_ref,
                  xf_ref, wf_ref, xm_ref, wb_ref, m_ref, xsem, wsem):
    D = xp_ref.shape[2]
    NS = xm_ref.shape[0] // TS
    NO = wb_ref.shape[0] // TO
    s = pl.program_id(0)
    o = pl.program_id(1)

    def x_dma(slab, buf):
        return pltpu.make_async_copy(
            x_any.at[0, pl.ds(slab * TS, TS), :], xf_ref.at[buf], xsem.at[buf])

    def w_dma(slab, buf):
        return pltpu.make_async_copy(
            w_any.at[pl.ds(slab * TO, TO), :], wf_ref.at[buf], wsem.at[buf])

    @pl.when(jnp.logical_and(s == 0, o == 0))
    def _():
        x_dma(0, 0).start()
        w_dma(0, 0).start()
        x_dma(1, 1).start()
        w_dma(1, 1).start()
        pb = pb_ref[0, 0]
        sel = _compute_sel(pb, xp_ref[0])
        _scatter_mask(sel, perm_ref[...].reshape(1, D), m_ref)

    @pl.when(o == 0)
    def _():
        sb = lax.rem(s, 2)
        x_dma(s, sb).wait()
        xm_ref[pl.ds(s * TS, TS), :] = (
            xf_ref[sb] * m_ref[...]).astype(jnp.bfloat16)

        @pl.when(s + 2 < NS)
        def _():
            x_dma(s + 2, sb).start()

    @pl.when(s == 0)
    def _():
        ob = lax.rem(o, 2)
        w_dma(o, ob).wait()
        wb_ref[pl.ds(o * TO, TO), :] = wf_ref[ob].astype(jnp.bfloat16)

        @pl.when(o + 2 < NO)
        def _():
            w_dma(o + 2, ob).start()

    out_ref[0] = lax.dot_general(
        xm_ref[pl.ds(s * TS, TS), :], wb_ref[pl.ds(o * TO, TO), :],
        (((1,), (1,)), ((), ())),
        preferred_element_type=jnp.float32)


def kernel(x, new_weight, permutation, prompt_boundary):
    _, S, D = x.shape
    O = new_weight.shape[0]
    pb = jnp.asarray(prompt_boundary, jnp.int32).reshape(1, 1)
    perm = permutation.astype(jnp.int32)
    return pl.pallas_call(
        _fused_kernel,
        grid=(S // TS, O // TO),
        in_specs=[
            pl.BlockSpec(memory_space=pltpu.SMEM),
            pl.BlockSpec((D,), lambda s, o: (0,)),
            pl.BlockSpec((1, PROMPT_WIN, D), lambda s, o: (0, 0, 0)),
            pl.BlockSpec(memory_space=pl.ANY),
            pl.BlockSpec(memory_space=pl.ANY),
        ],
        out_specs=pl.BlockSpec((1, TS, TO), lambda s, o: (0, s, o)),
        out_shape=jax.ShapeDtypeStruct((1, S, O), jnp.float32),
        scratch_shapes=[
            pltpu.VMEM((2, TS, D), jnp.float32),
            pltpu.VMEM((2, TO, D), jnp.float32),
            pltpu.VMEM((S, D), jnp.bfloat16),
            pltpu.VMEM((O, D), jnp.bfloat16),
            pltpu.VMEM((1, D), jnp.float32),
            pltpu.SemaphoreType.DMA((2,)),
            pltpu.SemaphoreType.DMA((2,)),
        ],
    )(pb, perm, x, x, new_weight)


# speedup vs baseline: 1.0756x; 1.0756x over previous
import jax
import jax.numpy as jnp
from jax import lax
from jax.experimental import pallas as pl
from jax.experimental.pallas import tpu as pltpu

TOPK = 512
PROMPT_WIN = 256
NHI = 16


def _compute_sel(pb, xs):
    D = xs.shape[1]
    rows = lax.broadcasted_iota(jnp.int32, (PROMPT_WIN, 1), 0)
    rmask = (rows <= pb).astype(jnp.float32)
    s = jnp.sum(xs * rmask, axis=0, keepdims=True)
    a = jnp.abs(s) / (pb + 1).astype(jnp.float32)
    ab = lax.bitcast_convert_type(a, jnp.int32)

    def bs_body(_, carry):
        lo, hi = carry
        mid = lo + (hi - lo) // 2
        ge = jnp.sum((ab >= mid).astype(jnp.int32))
        take = ge >= TOPK
        return jnp.where(take, mid, lo), jnp.where(take, hi, mid)

    lo, _ = lax.fori_loop(
        0, 31, bs_body, (jnp.int32(0), jnp.int32(0x7F800001)))
    v = lo
    gt = ab > v
    c_gt = jnp.sum(gt.astype(jnp.int32))
    r = TOPK - c_gt
    eq = ab == v
    idx = lax.broadcasted_iota(jnp.int32, (1, D), 1)

    def bs2_body(_, carry):
        lo2, hi2 = carry
        mid = lo2 + (hi2 - lo2) // 2
        g = jnp.sum((eq & (idx <= mid)).astype(jnp.int32))
        ok = g >= r
        lo2n = jnp.where(ok, lo2, mid)
        hi2n = jnp.where(ok, mid, hi2)
        valid = (hi2 - lo2) > 1
        return (jnp.where(valid, lo2n, lo2), jnp.where(valid, hi2n, hi2))

    _, I = lax.fori_loop(0, 11, bs2_body, (jnp.int32(-1), jnp.int32(D - 1)))
    return (gt | (eq & (idx <= I))).astype(jnp.float32)


def _fused_kernel(pb_ref, perm_ref, xp_ref, x_any, w_ref, out_ref,
                  xf_ref, xm_ref, m_ref, sem):
    D = xp_ref.shape[2]
    nlo = D // NHI

    S = xf_ref.shape[0]
    nck = 4
    cs = S // nck

    @pl.when(pl.program_id(0) == 0)
    def _():
        cps = [
            pltpu.make_async_copy(
                x_any.at[0, pl.ds(k * cs, cs), :],
                xf_ref.at[pl.ds(k * cs, cs), :], sem.at[k])
            for k in range(nck)
        ]
        for cp in cps:
            cp.start()
        pb = pb_ref[0, 0]
        sel = _compute_sel(pb, xp_ref[0])
        permc = jnp.transpose(perm_ref[...].reshape(1, D))
        selc = jnp.transpose(sel)
        hids = lax.broadcasted_iota(jnp.int32, (1, NHI), 1)
        lids = lax.broadcasted_iota(jnp.int32, (1, nlo), 1)
        asel = jnp.where((permc // nlo) == hids, selc, 0.0).astype(jnp.bfloat16)
        b = ((permc % nlo) == lids).astype(jnp.bfloat16)
        mm = lax.dot_general(
            asel, b, (((0,), (0,)), ((), ())),
            preferred_element_type=jnp.float32)
        for h in range(NHI):
            m_ref[0:1, h * nlo:(h + 1) * nlo] = mm[h:h + 1, :]
        for k, cp in enumerate(cps):
            cp.wait()
            xm_ref[pl.ds(k * cs, cs), :] = (
                xf_ref[pl.ds(k * cs, cs), :] * m_ref[...]
            ).astype(jnp.bfloat16)

    wb = w_ref[...].astype(jnp.bfloat16)
    out_ref[0] = lax.dot_general(
        xm_ref[...], wb, (((1,), (1,)), ((), ())),
        preferred_element_type=jnp.float32)


def kernel(x, new_weight, permutation, prompt_boundary, to=512):
    _, S, D = x.shape
    O = new_weight.shape[0]
    pb = jnp.asarray(prompt_boundary, jnp.int32).reshape(1, 1)
    perm = permutation.astype(jnp.int32)
    return pl.pallas_call(
        _fused_kernel,
        grid=(O // to,),
        in_specs=[
            pl.BlockSpec(memory_space=pltpu.SMEM),
            pl.BlockSpec((D,), lambda j: (0,)),
            pl.BlockSpec((1, PROMPT_WIN, D), lambda j: (0, 0, 0)),
            pl.BlockSpec(memory_space=pl.ANY),
            pl.BlockSpec((to, D), lambda j: (j, 0)),
        ],
        out_specs=pl.BlockSpec((1, S, to), lambda j: (0, 0, j)),
        out_shape=jax.ShapeDtypeStruct((1, S, O), jnp.float32),
        scratch_shapes=[
            pltpu.VMEM((S, D), jnp.float32),
            pltpu.VMEM((S, D), jnp.bfloat16),
            pltpu.VMEM((1, D), jnp.float32),
            pltpu.SemaphoreType.DMA((4,)),
        ],
    )(pb, perm, x, x, new_weight)
